# Initial kernel scaffold; baseline (speedup 1.0000x reference)
import jax
import jax.numpy as jnp
from jax.experimental import pallas as pl


def kernel(indices, codebook):
    raise NotImplementedError("write your pallas kernel here")


# speedup vs baseline: 23.7520x; 23.7520x over previous
import functools

import jax
import jax.numpy as jnp
from jax import lax
from jax.experimental import pallas as pl
from jax.experimental.pallas import tpu as pltpu
from jax.experimental.pallas import tpu_sc as plsc

_BATCH = 16384
_NUM_TOKENS = 32
_CODEBOOK_SIZE = 64
_HIDDEN = 128


@functools.lru_cache(maxsize=None)
def _build_sc_counts():
    info = plsc.get_sparse_core_info()
    nc, ns, lanes = info.num_cores, info.num_subcores, info.num_lanes
    nw = nc * ns
    rpw = _BATCH // nw
    idx_words = rpw * _NUM_TOKENS
    cnt_words = rpw * _CODEBOOK_SIZE

    mesh = plsc.VectorSubcoreMesh(core_axis_name="c", subcore_axis_name="s")

    @functools.partial(
        pl.kernel,
        out_type=jax.ShapeDtypeStruct((_BATCH * _CODEBOOK_SIZE,), jnp.float32),
        mesh=mesh,
        scratch_types=[
            pltpu.VMEM((idx_words,), jnp.int32),
            pltpu.VMEM((cnt_words,), jnp.float32),
        ],
        compiler_params=pltpu.CompilerParams(needs_layout_passes=False),
    )
    def sc_counts(idx_hbm, cnt_hbm, idx_v, cnt_v):
        wid = lax.axis_index("s") * nc + lax.axis_index("c")
        pltpu.sync_copy(idx_hbm.at[pl.ds(wid * idx_words, idx_words)], idx_v)

        zero = jnp.zeros((lanes,), jnp.float32)
        ones = jnp.ones((lanes,), jnp.float32)
        lane_iota = lax.iota(jnp.int32, lanes)

        def zero_chunk(i, carry):
            cnt_v[pl.ds(i * lanes, lanes)] = zero
            return carry

        lax.fori_loop(0, cnt_words // lanes, zero_chunk, 0)

        def hist_block(rb, carry):
            idx_base = rb * (lanes * _NUM_TOKENS)
            cnt_rows = rb * (lanes * _CODEBOOK_SIZE) + lane_iota * _CODEBOOK_SIZE
            for t in range(_NUM_TOKENS):
                addr = idx_base + (lane_iota * _NUM_TOKENS + t)
                iv = plsc.load_gather(idx_v, [addr])
                plsc.addupdate_scatter(cnt_v, [cnt_rows + iv], ones)
            return carry

        lax.fori_loop(0, rpw // lanes, hist_block, 0)

        pltpu.sync_copy(cnt_v, cnt_hbm.at[pl.ds(wid * cnt_words, cnt_words)])

    return sc_counts


def _mm_body(cnt_ref, cb_ref, out_ref):
    out_ref[...] = lax.dot_general(
        cnt_ref[...],
        cb_ref[...],
        (((1,), (0,)), ((), ())),
        preferred_element_type=jnp.float32,
        precision=lax.Precision.HIGHEST,
    )


def kernel(indices, codebook):
    counts = _build_sc_counts()(indices.reshape(-1))
    counts = counts.reshape(_BATCH, _CODEBOOK_SIZE)
    bm = 2048
    out = pl.pallas_call(
        _mm_body,
        grid=(_BATCH // bm,),
        in_specs=[
            pl.BlockSpec((bm, _CODEBOOK_SIZE), lambda i: (i, 0)),
            pl.BlockSpec((_CODEBOOK_SIZE, _HIDDEN), lambda i: (0, 0)),
        ],
        out_specs=pl.BlockSpec((bm, _HIDDEN), lambda i: (i, 0)),
        out_shape=jax.ShapeDtypeStruct((_BATCH, _HIDDEN), jnp.float32),
    )(counts, codebook)
    return out


# speedup vs baseline: 30.5822x; 1.2876x over previous
import functools

import jax
import jax.numpy as jnp
from jax import lax
from jax.experimental import pallas as pl
from jax.experimental.pallas import tpu as pltpu
from jax.experimental.pallas import tpu_sc as plsc

_BATCH = 16384
_NUM_TOKENS = 32
_CODEBOOK_SIZE = 64
_HIDDEN = 128


@functools.lru_cache(maxsize=None)
def _build_sc_counts():
    info = plsc.get_sparse_core_info()
    nc, ns, lanes = info.num_cores, info.num_subcores, info.num_lanes
    nw = nc * ns
    rpw = _BATCH // nw

    mesh = plsc.VectorSubcoreMesh(core_axis_name="c", subcore_axis_name="s")

    @functools.partial(
        pl.kernel,
        out_type=jax.ShapeDtypeStruct((_BATCH, _CODEBOOK_SIZE), jnp.float32),
        mesh=mesh,
        scratch_types=[
            pltpu.VMEM((rpw, _NUM_TOKENS), jnp.int32),
            pltpu.VMEM((rpw, _CODEBOOK_SIZE), jnp.float32),
        ],
        compiler_params=pltpu.CompilerParams(needs_layout_passes=False),
    )
    def sc_counts(idx_hbm, cnt_hbm, idx_v, cnt_v):
        wid = lax.axis_index("s") * nc + lax.axis_index("c")
        base = wid * rpw
        pltpu.sync_copy(idx_hbm.at[pl.ds(base, rpw)], idx_v)

        zero = jnp.zeros((lanes,), jnp.float32)
        ones = jnp.ones((lanes,), jnp.float32)
        lane_iota = lax.iota(jnp.int32, lanes)

        def hist_block(rb, carry):
            r0 = rb * lanes
            for r in range(lanes):
                for c in range(_CODEBOOK_SIZE // lanes):
                    cnt_v[r0 + r, pl.ds(c * lanes, lanes)] = zero
            rows = r0 + lane_iota
            for t in range(_NUM_TOKENS):
                tv = jnp.full((lanes,), t, jnp.int32)
                iv = plsc.load_gather(idx_v, [rows, tv])
                plsc.addupdate_scatter(cnt_v, [rows, iv], ones)
            return carry

        lax.fori_loop(0, rpw // lanes, hist_block, 0)

        pltpu.sync_copy(cnt_v, cnt_hbm.at[pl.ds(base, rpw)])

    return sc_counts


def _mm_body(cnt_ref, cb_ref, out_ref):
    out_ref[...] = lax.dot_general(
        cnt_ref[...],
        cb_ref[...],
        (((1,), (0,)), ((), ())),
        preferred_element_type=jnp.float32,
        precision=lax.Precision.HIGHEST,
    )


def kernel(indices, codebook):
    counts = _build_sc_counts()(indices)
    bm = 2048
    out = pl.pallas_call(
        _mm_body,
        grid=(_BATCH // bm,),
        in_specs=[
            pl.BlockSpec((bm, _CODEBOOK_SIZE), lambda i: (i, 0)),
            pl.BlockSpec((_CODEBOOK_SIZE, _HIDDEN), lambda i: (0, 0)),
        ],
        out_specs=pl.BlockSpec((bm, _HIDDEN), lambda i: (i, 0)),
        out_shape=jax.ShapeDtypeStruct((_BATCH, _HIDDEN), jnp.float32),
    )(counts, codebook)
    return out


# speedup vs baseline: 41.5731x; 1.3594x over previous
import functools

import jax
import jax.numpy as jnp
from jax import lax
from jax.experimental import pallas as pl
from jax.experimental.pallas import tpu as pltpu
from jax.experimental.pallas import tpu_sc as plsc

_BATCH = 16384
_NUM_TOKENS = 32
_CODEBOOK_SIZE = 64
_HIDDEN = 128


@functools.lru_cache(maxsize=None)
def _build_sc_counts():
    info = plsc.get_sparse_core_info()
    nc, ns, lanes = info.num_cores, info.num_subcores, info.num_lanes
    nw = nc * ns
    rpw = _BATCH // nw

    mesh = plsc.VectorSubcoreMesh(core_axis_name="c", subcore_axis_name="s")

    @functools.partial(
        pl.kernel,
        out_type=jax.ShapeDtypeStruct((_BATCH, _CODEBOOK_SIZE), jnp.float32),
        mesh=mesh,
        scratch_types=[
            pltpu.VMEM((rpw, _NUM_TOKENS), jnp.int32),
            pltpu.VMEM((rpw, _CODEBOOK_SIZE), jnp.float32),
        ],
        compiler_params=pltpu.CompilerParams(needs_layout_passes=False),
    )
    def sc_counts(idx_hbm, cnt_hbm, idx_v, cnt_v):
        wid = lax.axis_index("s") * nc + lax.axis_index("c")
        base = wid * rpw
        pltpu.sync_copy(idx_hbm.at[pl.ds(base, rpw)], idx_v)

        zero = jnp.zeros((lanes,), jnp.float32)
        ones = jnp.ones((lanes,), jnp.float32)

        def hist_row(r):
            rows = jnp.full((lanes,), r, jnp.int32)
            for c in range(_CODEBOOK_SIZE // lanes):
                cnt_v[r, pl.ds(c * lanes, lanes)] = zero
            for t0 in range(_NUM_TOKENS // lanes):
                iv = idx_v[r, pl.ds(t0 * lanes, lanes)]
                plsc.addupdate_scatter(cnt_v, [rows, iv], ones)

        plsc.parallel_loop(0, rpw, unroll=4)(hist_row)

        pltpu.sync_copy(cnt_v, cnt_hbm.at[pl.ds(base, rpw)])

    return sc_counts


def _mm_body(cnt_ref, cb_ref, out_ref):
    out_ref[...] = lax.dot_general(
        cnt_ref[...],
        cb_ref[...],
        (((1,), (0,)), ((), ())),
        preferred_element_type=jnp.float32,
        precision=lax.Precision.HIGHEST,
    )


def kernel(indices, codebook):
    counts = _build_sc_counts()(indices)
    bm = 2048
    out = pl.pallas_call(
        _mm_body,
        grid=(_BATCH // bm,),
        in_specs=[
            pl.BlockSpec((bm, _CODEBOOK_SIZE), lambda i: (i, 0)),
            pl.BlockSpec((_CODEBOOK_SIZE, _HIDDEN), lambda i: (0, 0)),
        ],
        out_specs=pl.BlockSpec((bm, _HIDDEN), lambda i: (i, 0)),
        out_shape=jax.ShapeDtypeStruct((_BATCH, _HIDDEN), jnp.float32),
    )(counts, codebook)
    return out


# speedup vs baseline: 44.1678x; 1.0624x over previous
import functools

import jax
import jax.numpy as jnp
from jax import lax
from jax.experimental import pallas as pl
from jax.experimental.pallas import tpu as pltpu
from jax.experimental.pallas import tpu_sc as plsc

_BATCH = 16384
_NUM_TOKENS = 32
_CODEBOOK_SIZE = 64
_HIDDEN = 128


@functools.lru_cache(maxsize=None)
def _build_sc_counts():
    info = plsc.get_sparse_core_info()
    nc, ns, lanes = info.num_cores, info.num_subcores, info.num_lanes
    nw = nc * ns
    rpw = _BATCH // nw

    mesh = plsc.VectorSubcoreMesh(core_axis_name="c", subcore_axis_name="s")

    @functools.partial(
        pl.kernel,
        out_type=jax.ShapeDtypeStruct((_BATCH, _CODEBOOK_SIZE), jnp.float32),
        mesh=mesh,
        scratch_types=[
            pltpu.VMEM((rpw, _NUM_TOKENS), jnp.int32),
            pltpu.VMEM((rpw, _CODEBOOK_SIZE), jnp.float32),
        ],
        compiler_params=pltpu.CompilerParams(
            needs_layout_passes=False, use_tc_tiling_on_sc=True
        ),
    )
    def sc_counts(idx_hbm, cnt_hbm, idx_v, cnt_v):
        wid = lax.axis_index("s") * nc + lax.axis_index("c")
        base = wid * rpw
        pltpu.sync_copy(idx_hbm.at[pl.ds(base, rpw)], idx_v)

        zero = jnp.zeros((lanes,), jnp.float32)
        ones = jnp.ones((lanes,), jnp.float32)

        def hist_row(r):
            rows = jnp.full((lanes,), r, jnp.int32)
            for c in range(_CODEBOOK_SIZE // lanes):
                cnt_v[r, pl.ds(c * lanes, lanes)] = zero
            for t0 in range(_NUM_TOKENS // lanes):
                iv = idx_v[r, pl.ds(t0 * lanes, lanes)]
                plsc.addupdate_scatter(cnt_v, [rows, iv], ones)

        plsc.parallel_loop(0, rpw, unroll=4)(hist_row)

        pltpu.sync_copy(cnt_v, cnt_hbm.at[pl.ds(base, rpw)])

    return sc_counts


def _mm_body(cnt_ref, cb_ref, out_ref):
    out_ref[...] = lax.dot_general(
        cnt_ref[...],
        cb_ref[...],
        (((1,), (0,)), ((), ())),
        preferred_element_type=jnp.float32,
        precision=lax.Precision.DEFAULT,
    )


def kernel(indices, codebook):
    counts = _build_sc_counts()(indices)
    bm = 2048
    out = pl.pallas_call(
        _mm_body,
        grid=(_BATCH // bm,),
        in_specs=[
            pl.BlockSpec((bm, _CODEBOOK_SIZE), lambda i: (i, 0)),
            pl.BlockSpec((_CODEBOOK_SIZE, _HIDDEN), lambda i: (0, 0)),
        ],
        out_specs=pl.BlockSpec((bm, _HIDDEN), lambda i: (i, 0)),
        out_shape=jax.ShapeDtypeStruct((_BATCH, _HIDDEN), jnp.float32),
    )(counts, codebook)
    return out
